# baseline (speedup 1.0000x reference)
import jax
import jax.numpy as jnp
from jax import lax
from jax.experimental import pallas as pl
from jax.experimental.pallas import tpu as pltpu
from jax.experimental.pallas import tpu_sc as plsc

_NC = 2
_NS = 16
_NW = _NC * _NS
_L = 16

_BATCH = 16384
_EMB_DIM = 16
_B_PER_W = _BATCH // _NW
_SUB = 8
_CHUNK = 16
_NCHUNK = _B_PER_W // _CHUNK


def _gather_body(y_hbm, table_hbm, out_hbm, idx_v, buf, rows_v, sems):
    wid = lax.axis_index("s") * _NC + lax.axis_index("c")
    base = wid * _B_PER_W
    pltpu.sync_copy(y_hbm.at[pl.ds(base, _B_PER_W)], idx_v)

    def fire(c, b):
        vec = idx_v[pl.ds(c * _CHUNK, _L)]
        bvec = (vec >> 3) << 3
        for lane in range(_L):
            pltpu.make_async_copy(
                table_hbm.at[pl.ds(pl.multiple_of(bvec[lane], _SUB), _SUB)],
                buf.at[b].at[lane],
                sems.at[b],
            ).start()

    def drain(b):
        pltpu.make_async_copy(
            table_hbm.at[pl.ds(0, _CHUNK * _SUB)],
            rows_v.at[pl.ds(0, _CHUNK * _SUB)],
            sems.at[b],
        ).wait()

    def extract(c, b):
        yv = idx_v[pl.ds(c * _CHUNK, _L)]
        for lane in range(_L):
            sub = yv[lane] & (_SUB - 1)
            rows_v[c * _CHUNK + lane] = buf[b, lane, sub]

    fire(0, 0)

    def step(p, _):
        c0 = 2 * p
        c1 = c0 + 1
        fire(c1, 1)
        drain(0)
        extract(c0, 0)

        @pl.when(p < _NCHUNK // 2 - 1)
        def _():
            fire(c0 + 2, 0)

        drain(1)
        extract(c1, 1)
        return ()

    lax.fori_loop(0, _NCHUNK // 2, step, ())

    pltpu.sync_copy(rows_v, out_hbm.at[pl.ds(base, _B_PER_W)])


@jax.jit
def _gather(y, emb_table):
    mesh = plsc.VectorSubcoreMesh(core_axis_name="c", subcore_axis_name="s")
    kern = pl.kernel(
        _gather_body,
        out_type=jax.ShapeDtypeStruct((_BATCH, _EMB_DIM), jnp.float32),
        mesh=mesh,
        scratch_types=[
            pltpu.VMEM((_B_PER_W,), jnp.int32),
            pltpu.VMEM((2, _CHUNK, _SUB, _EMB_DIM), jnp.float32),
            pltpu.VMEM((_B_PER_W, _EMB_DIM), jnp.float32),
            pltpu.SemaphoreType.DMA((2,)),
        ],
    )
    return kern(y, emb_table)


def kernel(y, emb_table):
    return _gather(y.astype(jnp.int32), emb_table)
